# Initial kernel scaffold; baseline (speedup 1.0000x reference)
import jax
import jax.numpy as jnp
from jax.experimental import pallas as pl


def kernel(x, y, xconv_w, xconv_b, yconv_w, yconv_b, out_w, out_b, ln_w, ln_b):
    raise NotImplementedError("write your pallas kernel here")


# speedup vs baseline: 7.9848x; 7.9848x over previous
import functools

import jax
import jax.numpy as jnp
from jax import lax
from jax.experimental import pallas as pl
from jax.experimental.pallas import tpu as pltpu
from jax.experimental.pallas import tpu_sc as plsc

C = 64
H = W = 224
BS = 28
NB = 8
NBLK = NB * NB
HWB = BS * BS
TOPK = 5
WP = 256
RP = 252
BAND = BS * WP
ROWF = C * HWB
NCH = 8
CH = ROWF // NCH
SL = CH // 128


def _conv_kernel(w_ref, b_ref, wv_ref, cur_ref, nxt_ref, out_ref, s_ref):
    xcat = jnp.concatenate([cur_ref[0], nxt_ref[0]], axis=1)
    acc = jnp.zeros((C, BAND), jnp.float32)
    for dy in range(3):
        win = xcat[:, dy * WP: dy * WP + BAND]
        for dx in (-1, 0, 1):
            xs = win if dx == 0 else jnp.roll(win, -dx, axis=1)
            acc = acc + jnp.dot(w_ref[0, dy * 3 + dx + 1], xs,
                                preferred_element_type=jnp.float32)
    acc = jnp.maximum(acc + b_ref[0, 0][:, None], 0.0)
    out_ref[0] = acc
    s_ref[0, 0] = jnp.dot(wv_ref[0], acc, preferred_element_type=jnp.float32)


def _run_conv(imgs, wt, bias, wvec):
    G = imgs.shape[0]
    return pl.pallas_call(
        _conv_kernel,
        grid=(G, NB),
        in_specs=[
            pl.BlockSpec((1, 9, C, C), lambda g, i: (g, 0, 0, 0)),
            pl.BlockSpec((1, 1, C), lambda g, i: (g, 0, 0)),
            pl.BlockSpec((1, 1, C), lambda g, i: (g, 0, 0)),
            pl.BlockSpec((1, C, BAND), lambda g, i: (g, 0, i)),
            pl.BlockSpec((1, C, BAND), lambda g, i: (g, 0, i + 1)),
        ],
        out_specs=[
            pl.BlockSpec((1, C, BAND), lambda g, i: (g, 0, i)),
            pl.BlockSpec((1, 1, 1, BAND), lambda g, i: (g, i, 0, 0)),
        ],
        out_shape=[
            jax.ShapeDtypeStruct((G, C, NB * BAND), jnp.float32),
            jax.ShapeDtypeStruct((G, NB, 1, BAND), jnp.float32),
        ],
    )(wt, bias, wvec, imgs, imgs)


def _score_kernel(sx_ref, sy_ref, idx_ref):
    sxv = sx_ref[...]
    syv = sy_ref[...]
    ones = jnp.ones((1, HWB), jnp.float32)
    sy_mean = lax.dot_general(ones, syv, (((1,), (1,)), ((), ())),
                              preferred_element_type=jnp.float32) / HWB
    lane = lax.broadcasted_iota(jnp.int32, (NBLK, NBLK), 1)

    def pair_body(l, sc):
        syl = sy_ref[pl.ds(l, 1), :]
        m = jnp.max(sxv + syl, axis=1, keepdims=True)
        return jnp.where(lane == l, m, sc)

    score = lax.fori_loop(0, NBLK, pair_body,
                          jnp.full((NBLK, NBLK), -1e30, jnp.float32))
    score = score + sy_mean
    lane16 = lax.broadcasted_iota(jnp.int32, (NBLK, 16), 1)

    def top_body(k, carry):
        sc, idxs = carry
        mx = jnp.max(sc, axis=1, keepdims=True)
        am = jnp.min(jnp.where(sc == mx, lane, NBLK), axis=1, keepdims=True)
        sc = jnp.where(lane == am, -1e30, sc)
        idxs = jnp.where(lane16 == k, am, idxs)
        return sc, idxs

    _, idxs = lax.fori_loop(0, TOPK, top_body,
                            (score, jnp.zeros((NBLK, 16), jnp.int32)))
    idx_ref[...] = idxs


def _gsum_sc(ytab, idx16):
    mesh = plsc.VectorSubcoreMesh(core_axis_name="c", subcore_axis_name="s")

    @functools.partial(
        pl.kernel,
        mesh=mesh,
        out_type=jax.ShapeDtypeStruct((NBLK, ROWF), jnp.float32),
        scratch_types=[
            pltpu.VMEM((NBLK, 16), jnp.int32),
            pltpu.VMEM((16,), jnp.int32),
            pltpu.VMEM((TOPK, SL, 128), jnp.float32),
            pltpu.VMEM((CH,), jnp.float32),
            pltpu.SemaphoreType.DMA,
        ],
    )
    def gsum(ytab_hbm, idx_hbm, out_hbm, idxall, iv, buf, acc, sem):
        wid = lax.axis_index("s") * 2 + lax.axis_index("c")
        pltpu.sync_copy(idx_hbm, idxall)

        def do_ij(r, carry):
            ij = wid * 2 + r
            ivec = idxall[ij, :] * NCH

            def do_chunk(c, carry2):
                iv[...] = ivec + c
                pltpu.async_copy(ytab_hbm.at[iv.at[pl.ds(0, TOPK)]],
                                 buf, sem).wait()

                def add_body(t, carry3):
                    s = t // 8
                    o = (t % 8) * 16
                    v = (buf[0, s, pl.ds(o, 16)] + buf[1, s, pl.ds(o, 16)]
                         + buf[2, s, pl.ds(o, 16)] + buf[3, s, pl.ds(o, 16)]
                         + buf[4, s, pl.ds(o, 16)])
                    acc[pl.ds(t * 16, 16)] = v
                    return carry3

                lax.fori_loop(0, CH // 16, add_body, 0)
                pltpu.sync_copy(acc, out_hbm.at[ij, pl.ds(c * CH, CH)])
                return carry2

            lax.fori_loop(0, NCH, do_chunk, 0)
            return carry

        lax.fori_loop(0, 2, do_ij, 0)

    return gsum(ytab, idx16)


def _mm_kernel(xb_ref, ys_ref, out_ref):
    g = lax.dot_general(xb_ref[0], ys_ref[0], (((1,), (1,)), ((), ())),
                        preferred_element_type=jnp.float32)
    out_ref[0] = lax.dot_general(g, xb_ref[0], (((1,), (0,)), ((), ())),
                                 preferred_element_type=jnp.float32)


def _mm_tc(xblk, ysum3):
    return pl.pallas_call(
        _mm_kernel,
        grid=(NBLK,),
        in_specs=[
            pl.BlockSpec((1, C, HWB), lambda ij: (ij, 0, 0)),
            pl.BlockSpec((1, C, HWB), lambda ij: (ij, 0, 0)),
        ],
        out_specs=pl.BlockSpec((1, C, HWB), lambda ij: (ij, 0, 0)),
        out_shape=jax.ShapeDtypeStruct((NBLK, C, HWB), jnp.float32),
    )(xblk, ysum3)


def _pad_flat(img):
    p = jnp.pad(img, ((0, 0), (0, 0), (1, RP - H - 1), (1, WP - W - 1)))
    return p.reshape(img.shape[0], C, RP * WP)


def _blockify(img):
    return img.reshape(C, NB, BS, NB, BS).transpose(1, 3, 0, 2, 4).reshape(NBLK, C, HWB)


def kernel(x, y, xconv_w, xconv_b, yconv_w, yconv_b, out_w, out_b, ln_w, ln_b):
    imgs = _pad_flat(jnp.stack([x[0], y[0]]))
    wt = jnp.stack([xconv_w, yconv_w]).transpose(0, 3, 4, 1, 2).reshape(2, 9, C, C)
    bias = jnp.stack([xconv_b, yconv_b]).reshape(2, 1, C)
    wvec = jnp.stack([ln_w[0, C:], ln_w[0, :C]]).reshape(2, 1, C)

    conv, smap = _run_conv(imgs, wt, bias, wvec)
    co = conv.reshape(2, C, H, WP)[:, :, :, 1:W + 1]
    sm = smap.reshape(2, H, WP)[:, :, 1:W + 1]
    sblk = sm.reshape(2, NB, BS, NB, BS).transpose(0, 1, 3, 2, 4).reshape(2, NBLK, HWB)

    idx16 = pl.pallas_call(
        _score_kernel,
        out_shape=jax.ShapeDtypeStruct((NBLK, 16), jnp.int32),
    )(sblk[0], sblk[1])

    xblk = _blockify(co[0])
    yblk = _blockify(co[1])
    ytab = yblk.reshape(NBLK * NCH, SL, 128)
    ysum = _gsum_sc(ytab, idx16)
    ob = _mm_tc(xblk, ysum.reshape(NBLK, C, HWB))

    oimg = ob.reshape(NB, NB, C, BS, BS).transpose(2, 0, 3, 1, 4).reshape(1, C, H, W)
    wt3 = out_w.transpose(2, 3, 0, 1).reshape(1, 9, C, C)
    conv3, _ = _run_conv(_pad_flat(oimg), wt3, out_b.reshape(1, 1, C),
                         jnp.zeros((1, 1, C), jnp.float32))
    return conv3.reshape(1, C, H, WP)[:, :, :, 1:W + 1]


# speedup vs baseline: 9.3775x; 1.1744x over previous
import functools

import jax
import jax.numpy as jnp
from jax import lax
from jax.experimental import pallas as pl
from jax.experimental.pallas import tpu as pltpu
from jax.experimental.pallas import tpu_sc as plsc

C = 64
H = W = 224
BS = 28
NB = 8
NBLK = NB * NB
HWB = BS * BS
TOPK = 5
WP = 256
RP = 252
BAND = BS * WP
ROWF = C * HWB
NCH = 8
CH = ROWF // NCH
SL = CH // 128


def _conv_taps(w_ref, b_ref, cur_ref, nxt_ref):
    xcat = jnp.concatenate([cur_ref[0], nxt_ref[0]], axis=1)
    acc = jnp.zeros((C, BAND), jnp.float32)
    for dy in range(3):
        win = xcat[:, dy * WP: dy * WP + BAND]
        for dx in (-1, 0, 1):
            xs = win if dx == 0 else jnp.roll(win, -dx, axis=1)
            acc = acc + jnp.dot(w_ref[0, dy * 3 + dx + 1], xs,
                                preferred_element_type=jnp.float32)
    return jnp.maximum(acc + b_ref[0, 0][:, None], 0.0)


def _conv_blk_kernel(w_ref, b_ref, wv_ref, cur_ref, nxt_ref, out_ref, s_ref):
    acc = _conv_taps(w_ref, b_ref, cur_ref, nxt_ref)
    acc = jnp.roll(acc, -1, axis=1)
    s = jnp.dot(wv_ref[0], acc, preferred_element_type=jnp.float32)
    acc3 = acc.reshape(C, BS, WP)
    s3 = s.reshape(1, BS, WP)
    for j in range(NB):
        out_ref[0, 0, j] = acc3[:, :, BS * j: BS * (j + 1)].reshape(C, HWB)
        s_ref[0, 0, j, 0] = s3[:, :, BS * j: BS * (j + 1)].reshape(HWB)


def _run_conv_blk(imgs, wt, bias, wvec):
    return pl.pallas_call(
        _conv_blk_kernel,
        grid=(2, NB),
        in_specs=[
            pl.BlockSpec((1, 9, C, C), lambda g, i: (g, 0, 0, 0)),
            pl.BlockSpec((1, 1, C), lambda g, i: (g, 0, 0)),
            pl.BlockSpec((1, 1, C), lambda g, i: (g, 0, 0)),
            pl.BlockSpec((1, C, BAND), lambda g, i: (g, 0, i)),
            pl.BlockSpec((1, C, BAND), lambda g, i: (g, 0, i + 1)),
        ],
        out_specs=[
            pl.BlockSpec((1, 1, NB, C, HWB), lambda g, i: (g, i, 0, 0, 0)),
            pl.BlockSpec((1, 1, NB, 1, HWB), lambda g, i: (g, i, 0, 0, 0)),
        ],
        out_shape=[
            jax.ShapeDtypeStruct((2, NB, NB, C, HWB), jnp.float32),
            jax.ShapeDtypeStruct((2, NB, NB, 1, HWB), jnp.float32),
        ],
    )(wt, bias, wvec, imgs, imgs)


def _conv_flat_kernel(w_ref, b_ref, cur_ref, nxt_ref, out_ref):
    out_ref[0] = _conv_taps(w_ref, b_ref, cur_ref, nxt_ref)


def _run_conv_flat(imgs, wt, bias):
    return pl.pallas_call(
        _conv_flat_kernel,
        grid=(1, NB),
        in_specs=[
            pl.BlockSpec((1, 9, C, C), lambda g, i: (g, 0, 0, 0)),
            pl.BlockSpec((1, 1, C), lambda g, i: (g, 0, 0)),
            pl.BlockSpec((1, C, BAND), lambda g, i: (g, 0, i)),
            pl.BlockSpec((1, C, BAND), lambda g, i: (g, 0, i + 1)),
        ],
        out_specs=pl.BlockSpec((1, C, BAND), lambda g, i: (g, 0, i)),
        out_shape=jax.ShapeDtypeStruct((1, C, NB * BAND), jnp.float32),
    )(wt, bias, imgs, imgs)


def _score_kernel(s_ref, idx_ref):
    sxv = s_ref[0]
    syv = s_ref[1]
    ones = jnp.ones((1, HWB), jnp.float32)
    sy_mean = lax.dot_general(ones, syv, (((1,), (1,)), ((), ())),
                              preferred_element_type=jnp.float32) / HWB
    lane = lax.broadcasted_iota(jnp.int32, (NBLK, NBLK), 1)

    def pair_body(l, sc):
        syl = s_ref[1, pl.ds(l, 1), :]
        m = jnp.max(sxv + syl, axis=1, keepdims=True)
        return jnp.where(lane == l, m, sc)

    score = lax.fori_loop(0, NBLK, pair_body,
                          jnp.full((NBLK, NBLK), -1e30, jnp.float32))
    score = score + sy_mean
    lane16 = lax.broadcasted_iota(jnp.int32, (NBLK, 16), 1)

    def top_body(k, carry):
        sc, idxs = carry
        mx = jnp.max(sc, axis=1, keepdims=True)
        am = jnp.min(jnp.where(sc == mx, lane, NBLK), axis=1, keepdims=True)
        sc = jnp.where(lane == am, -1e30, sc)
        idxs = jnp.where(lane16 == k, am, idxs)
        return sc, idxs

    _, idxs = lax.fori_loop(0, TOPK, top_body,
                            (score, jnp.zeros((NBLK, 16), jnp.int32)))
    idx_ref[...] = idxs


def _gsum_sc(xytab, idx16):
    mesh = plsc.VectorSubcoreMesh(core_axis_name="c", subcore_axis_name="s")

    @functools.partial(
        pl.kernel,
        mesh=mesh,
        out_type=jax.ShapeDtypeStruct((NBLK, ROWF), jnp.float32),
        scratch_types=[
            pltpu.VMEM((NBLK, 16), jnp.int32),
            pltpu.VMEM((16,), jnp.int32),
            pltpu.VMEM((TOPK, SL, 128), jnp.float32),
            pltpu.VMEM((CH,), jnp.float32),
            pltpu.SemaphoreType.DMA,
        ],
    )
    def gsum(ytab_hbm, idx_hbm, out_hbm, idxall, iv, buf, acc, sem):
        wid = lax.axis_index("s") * 2 + lax.axis_index("c")
        pltpu.sync_copy(idx_hbm, idxall)

        def do_ij(r, carry):
            ij = wid * 2 + r
            ivec = (idxall[ij, :] + NBLK) * NCH

            def do_chunk(c, carry2):
                iv[...] = ivec + c
                pltpu.async_copy(ytab_hbm.at[iv.at[pl.ds(0, TOPK)]],
                                 buf, sem).wait()

                def add_body(t, carry3):
                    s = t // 8
                    o = (t % 8) * 16
                    v = (buf[0, s, pl.ds(o, 16)] + buf[1, s, pl.ds(o, 16)]
                         + buf[2, s, pl.ds(o, 16)] + buf[3, s, pl.ds(o, 16)]
                         + buf[4, s, pl.ds(o, 16)])
                    acc[pl.ds(t * 16, 16)] = v
                    return carry3

                lax.fori_loop(0, CH // 16, add_body, 0)
                pltpu.sync_copy(acc, out_hbm.at[ij, pl.ds(c * CH, CH)])
                return carry2

            lax.fori_loop(0, NCH, do_chunk, 0)
            return carry

        lax.fori_loop(0, 2, do_ij, 0)

    return gsum(xytab, idx16)


def _mm_kernel(xy_ref, ys_ref, out_ref):
    xb = xy_ref[0, 0]
    g = lax.dot_general(xb, ys_ref[0], (((1,), (1,)), ((), ())),
                        preferred_element_type=jnp.float32)
    out_ref[0] = lax.dot_general(g, xb, (((1,), (0,)), ((), ())),
                                 preferred_element_type=jnp.float32)


def _mm_tc(xyblk, ysum3):
    return pl.pallas_call(
        _mm_kernel,
        grid=(NBLK,),
        in_specs=[
            pl.BlockSpec((1, 1, C, HWB), lambda ij: (0, ij, 0, 0)),
            pl.BlockSpec((1, C, HWB), lambda ij: (ij, 0, 0)),
        ],
        out_specs=pl.BlockSpec((1, C, HWB), lambda ij: (ij, 0, 0)),
        out_shape=jax.ShapeDtypeStruct((NBLK, C, HWB), jnp.float32),
    )(xyblk, ysum3)


def _pad_flat(img):
    p = jnp.pad(img, ((0, 0), (0, 0), (1, RP - H - 1), (1, WP - W - 1)))
    return p.reshape(img.shape[0], C, RP * WP)


def kernel(x, y, xconv_w, xconv_b, yconv_w, yconv_b, out_w, out_b, ln_w, ln_b):
    imgs = _pad_flat(jnp.stack([x[0], y[0]]))
    wt = jnp.stack([xconv_w, yconv_w]).transpose(0, 3, 4, 1, 2).reshape(2, 9, C, C)
    bias = jnp.stack([xconv_b, yconv_b]).reshape(2, 1, C)
    wvec = jnp.stack([ln_w[0, C:], ln_w[0, :C]]).reshape(2, 1, C)

    xyblk, sblk5 = _run_conv_blk(imgs, wt, bias, wvec)
    xyblk = xyblk.reshape(2, NBLK, C, HWB)

    idx16 = pl.pallas_call(
        _score_kernel,
        out_shape=jax.ShapeDtypeStruct((NBLK, 16), jnp.int32),
    )(sblk5.reshape(2, NBLK, HWB))

    ysum = _gsum_sc(xyblk.reshape(2 * NBLK * NCH, SL, 128), idx16)
    ob = _mm_tc(xyblk, ysum.reshape(NBLK, C, HWB))

    oimg = ob.reshape(NB, NB, C, BS, BS).transpose(2, 0, 3, 1, 4).reshape(1, C, H, W)
    wt3 = out_w.transpose(2, 3, 0, 1).reshape(1, 9, C, C)
    conv3 = _run_conv_flat(_pad_flat(oimg), wt3, out_b.reshape(1, 1, C))
    return conv3.reshape(1, C, H, WP)[:, :, :, 1:W + 1]


# speedup vs baseline: 10.6955x; 1.1405x over previous
import functools

import jax
import jax.numpy as jnp
from jax import lax
from jax.experimental import pallas as pl
from jax.experimental.pallas import tpu as pltpu
from jax.experimental.pallas import tpu_sc as plsc

C = 64
H = W = 224
BS = 28
NB = 8
NBLK = NB * NB
HWB = BS * BS
TOPK = 5
WP = 256
RP = 252
BAND = BS * WP
ROWF = C * HWB
NCH = 8
CH = ROWF // NCH
SL = CH // 128


def _conv_taps(w_ref, b_ref, cur_ref, nxt_ref):
    xcat = jnp.concatenate([cur_ref[0], nxt_ref[0]], axis=1)
    acc = jnp.zeros((C, BAND), jnp.float32)
    for dy in range(3):
        win = xcat[:, dy * WP: dy * WP + BAND]
        for dx in (-1, 0, 1):
            xs = win if dx == 0 else jnp.roll(win, -dx, axis=1)
            acc = acc + jnp.dot(w_ref[0, dy * 3 + dx + 1], xs,
                                preferred_element_type=jnp.float32)
    return jnp.maximum(acc + b_ref[0, 0][:, None], 0.0)


def _conv_blk_kernel(w_ref, b_ref, wv_ref, cur_ref, nxt_ref, out_ref, s_ref):
    acc = _conv_taps(w_ref, b_ref, cur_ref, nxt_ref)
    acc = jnp.roll(acc, -1, axis=1)
    s = jnp.dot(wv_ref[0], acc, preferred_element_type=jnp.float32)
    acc3 = acc.reshape(C, BS, WP)
    s3 = s.reshape(1, BS, WP)
    for j in range(NB):
        out_ref[0, j] = acc3[:, :, BS * j: BS * (j + 1)].reshape(C, HWB)
        s_ref[0, j] = s3[:, :, BS * j: BS * (j + 1)].reshape(HWB)


def _run_conv_blk(imgs, wt, bias, wvec):
    return pl.pallas_call(
        _conv_blk_kernel,
        grid=(2, NB),
        in_specs=[
            pl.BlockSpec((1, 9, C, C), lambda g, i: (g, 0, 0, 0)),
            pl.BlockSpec((1, 1, C), lambda g, i: (g, 0, 0)),
            pl.BlockSpec((1, 1, C), lambda g, i: (g, 0, 0)),
            pl.BlockSpec((1, C, BAND), lambda g, i: (g, 0, i)),
            pl.BlockSpec((1, C, BAND), lambda g, i: (g, 0, i + 1)),
        ],
        out_specs=[
            pl.BlockSpec((1, NB, C, HWB), lambda g, i: (g, i, 0, 0)),
            pl.BlockSpec((1, NB, HWB), lambda g, i: (g, i, 0)),
        ],
        out_shape=[
            jax.ShapeDtypeStruct((2, NBLK, C, HWB), jnp.float32),
            jax.ShapeDtypeStruct((2, NBLK, HWB), jnp.float32),
        ],
    )(wt, bias, wvec, imgs, imgs)


def _conv_flat_kernel(w_ref, b_ref, cur_ref, nxt_ref, out_ref):
    out_ref[0] = _conv_taps(w_ref, b_ref, cur_ref, nxt_ref)


def _run_conv_flat(imgs, wt, bias):
    return pl.pallas_call(
        _conv_flat_kernel,
        grid=(1, NB),
        in_specs=[
            pl.BlockSpec((1, 9, C, C), lambda g, i: (g, 0, 0, 0)),
            pl.BlockSpec((1, 1, C), lambda g, i: (g, 0, 0)),
            pl.BlockSpec((1, C, BAND), lambda g, i: (g, 0, i)),
            pl.BlockSpec((1, C, BAND), lambda g, i: (g, 0, i + 1)),
        ],
        out_specs=pl.BlockSpec((1, C, BAND), lambda g, i: (g, 0, i)),
        out_shape=jax.ShapeDtypeStruct((1, C, NB * BAND), jnp.float32),
    )(wt, bias, imgs, imgs)


def _score_kernel(s_ref, idx_ref):
    sxv = s_ref[0]
    syv = s_ref[1]
    ones = jnp.ones((1, HWB), jnp.float32)
    sy_mean = lax.dot_general(ones, syv, (((1,), (1,)), ((), ())),
                              preferred_element_type=jnp.float32) / HWB
    lane = lax.broadcasted_iota(jnp.int32, (NBLK, NBLK), 1)

    def pair_body(l, sc):
        syl = s_ref[1, pl.ds(l, 1), :]
        m = jnp.max(sxv + syl, axis=1, keepdims=True)
        return jnp.where(lane == l, m, sc)

    score = lax.fori_loop(0, NBLK, pair_body,
                          jnp.full((NBLK, NBLK), -1e30, jnp.float32))
    score = score + sy_mean
    lane16 = lax.broadcasted_iota(jnp.int32, (NBLK, 16), 1)

    def top_body(k, carry):
        sc, idxs = carry
        mx = jnp.max(sc, axis=1, keepdims=True)
        am = jnp.min(jnp.where(sc == mx, lane, NBLK), axis=1, keepdims=True)
        sc = jnp.where(lane == am, -1e30, sc)
        idxs = jnp.where(lane16 == k, am, idxs)
        return sc, idxs

    _, idxs = lax.fori_loop(0, TOPK, top_body,
                            (score, jnp.zeros((NBLK, 16), jnp.int32)))
    idx_ref[...] = idxs


def _gsum_sc(xytab, idx16):
    mesh = plsc.VectorSubcoreMesh(core_axis_name="c", subcore_axis_name="s")
    ntask = 2 * NCH

    @functools.partial(
        pl.kernel,
        mesh=mesh,
        out_type=jax.ShapeDtypeStruct((NBLK, C, HWB), jnp.float32),
        scratch_types=[
            pltpu.VMEM((NBLK, 16), jnp.int32),
            pltpu.VMEM((2, 16), jnp.int32),
            pltpu.VMEM((2, TOPK, SL, 128), jnp.float32),
            pltpu.VMEM((8, HWB), jnp.float32),
            pltpu.SemaphoreType.DMA,
            pltpu.SemaphoreType.DMA,
        ],
    )
    def gsum(ytab_hbm, idx_hbm, out_hbm, idxall, iv, buf, acc, sem0, sem1):
        wid = lax.axis_index("s") * 2 + lax.axis_index("c")
        pltpu.sync_copy(idx_hbm, idxall)
        ivecs = [(idxall[wid * 2 + r, :] + NBLK) * NCH for r in range(2)]
        sems = [sem0, sem1]

        def launch(t, slot):
            r, c = divmod(t, NCH)
            iv[slot, :] = ivecs[r] + c
            return pltpu.async_copy(
                ytab_hbm.at[iv.at[slot, pl.ds(0, TOPK)]],
                buf.at[slot], sems[slot])

        handles = {0: launch(0, 0), 1: launch(1, 1)}
        for t in range(ntask):
            slot = t % 2
            r, c = divmod(t, NCH)
            handles.pop(t).wait()

            def add_body(tt, carry):
                ch = tt // 49
                col = (tt % 49) * 16
                s = (tt * 16) // 128
                o = (tt * 16) % 128
                v = (buf[slot, 0, s, pl.ds(o, 16)]
                     + buf[slot, 1, s, pl.ds(o, 16)]
                     + buf[slot, 2, s, pl.ds(o, 16)]
                     + buf[slot, 3, s, pl.ds(o, 16)]
                     + buf[slot, 4, s, pl.ds(o, 16)])
                acc[ch, pl.ds(col, 16)] = v
                return carry

            lax.fori_loop(0, CH // 16, add_body, 0)
            pltpu.sync_copy(acc, out_hbm.at[wid * 2 + r, pl.ds(c * 8, 8)])
            if t + 2 < ntask:
                handles[t + 2] = launch(t + 2, slot)

    return gsum(xytab, idx16)


def _mm_kernel(xy_ref, ys_ref, out_ref):
    for b in range(4):
        xb = xy_ref[0, b]
        g = lax.dot_general(xb, ys_ref[b], (((1,), (1,)), ((), ())),
                            preferred_element_type=jnp.float32)
        out_ref[b] = lax.dot_general(g, xb, (((1,), (0,)), ((), ())),
                                     preferred_element_type=jnp.float32)


def _mm_tc(xyblk, ysum3):
    return pl.pallas_call(
        _mm_kernel,
        grid=(NBLK // 4,),
        in_specs=[
            pl.BlockSpec((1, 4, C, HWB), lambda b4: (0, b4, 0, 0)),
            pl.BlockSpec((4, C, HWB), lambda b4: (b4, 0, 0)),
        ],
        out_specs=pl.BlockSpec((4, C, HWB), lambda b4: (b4, 0, 0)),
        out_shape=jax.ShapeDtypeStruct((NBLK, C, HWB), jnp.float32),
    )(xyblk, ysum3)


def _pad_flat(img):
    p = jnp.pad(img, ((0, 0), (0, 0), (1, RP - H - 1), (1, WP - W - 1)))
    return p.reshape(img.shape[0], C, RP * WP)


def kernel(x, y, xconv_w, xconv_b, yconv_w, yconv_b, out_w, out_b, ln_w, ln_b):
    imgs = _pad_flat(jnp.stack([x[0], y[0]]))
    wt = jnp.stack([xconv_w, yconv_w]).transpose(0, 3, 4, 1, 2).reshape(2, 9, C, C)
    bias = jnp.stack([xconv_b, yconv_b]).reshape(2, 1, C)
    wvec = jnp.stack([ln_w[0, C:], ln_w[0, :C]]).reshape(2, 1, C)

    xyblk, sblk = _run_conv_blk(imgs, wt, bias, wvec)

    idx16 = pl.pallas_call(
        _score_kernel,
        out_shape=jax.ShapeDtypeStruct((NBLK, 16), jnp.int32),
    )(sblk)

    ysum = _gsum_sc(xyblk.reshape(2 * NBLK * NCH, SL, 128), idx16)
    ob = _mm_tc(xyblk, ysum)

    oimg = ob.reshape(NB, NB, C, BS, BS).transpose(2, 0, 3, 1, 4).reshape(1, C, H, W)
    wt3 = out_w.transpose(2, 3, 0, 1).reshape(1, 9, C, C)
    conv3 = _run_conv_flat(_pad_flat(oimg), wt3, out_b.reshape(1, 1, C))
    return conv3.reshape(1, C, H, WP)[:, :, :, 1:W + 1]


# speedup vs baseline: 10.9235x; 1.0213x over previous
import functools

import jax
import jax.numpy as jnp
from jax import lax
from jax.experimental import pallas as pl
from jax.experimental.pallas import tpu as pltpu
from jax.experimental.pallas import tpu_sc as plsc

C = 64
H = W = 224
BS = 28
NB = 8
NBLK = NB * NB
HWB = BS * BS
TOPK = 5
WP = 256
RP = 252
BAND = BS * WP
ROWF = C * HWB
NCH = 8
CH = ROWF // NCH
SL = CH // 128


def _conv_taps(w_ref, b_ref, cur_ref, nxt_ref):
    xcat = jnp.concatenate([cur_ref[0], nxt_ref[0]], axis=1)
    acc = jnp.zeros((C, BAND), jnp.float32)
    for dy in range(3):
        win = xcat[:, dy * WP: dy * WP + BAND]
        for dx in (-1, 0, 1):
            xs = win if dx == 0 else jnp.roll(win, -dx, axis=1)
            acc = acc + jnp.dot(w_ref[0, dy * 3 + dx + 1], xs,
                                preferred_element_type=jnp.float32)
    return jnp.maximum(acc + b_ref[0, 0][:, None], 0.0)


def _conv_blk_kernel(w_ref, b_ref, wv_ref, cur_ref, nxt_ref, out_ref, s_ref):
    acc = _conv_taps(w_ref, b_ref, cur_ref, nxt_ref)
    acc = jnp.roll(acc, -1, axis=1)
    s = jnp.dot(wv_ref[0], acc, preferred_element_type=jnp.float32)
    acc3 = acc.reshape(C, BS, WP)
    s3 = s.reshape(1, BS, WP)
    for j in range(NB):
        out_ref[0, j] = acc3[:, :, BS * j: BS * (j + 1)].reshape(C, HWB)
        s_ref[0, j] = s3[:, :, BS * j: BS * (j + 1)].reshape(HWB)


def _run_conv_blk(imgs, wt, bias, wvec):
    return pl.pallas_call(
        _conv_blk_kernel,
        grid=(2, NB),
        in_specs=[
            pl.BlockSpec((1, 9, C, C), lambda g, i: (g, 0, 0, 0)),
            pl.BlockSpec((1, 1, C), lambda g, i: (g, 0, 0)),
            pl.BlockSpec((1, 1, C), lambda g, i: (g, 0, 0)),
            pl.BlockSpec((1, C, BAND), lambda g, i: (g, 0, i)),
            pl.BlockSpec((1, C, BAND), lambda g, i: (g, 0, i + 1)),
        ],
        out_specs=[
            pl.BlockSpec((1, NB, C, HWB), lambda g, i: (g, i, 0, 0)),
            pl.BlockSpec((1, NB, HWB), lambda g, i: (g, i, 0)),
        ],
        out_shape=[
            jax.ShapeDtypeStruct((2, NBLK, C, HWB), jnp.float32),
            jax.ShapeDtypeStruct((2, NBLK, HWB), jnp.float32),
        ],
    )(wt, bias, wvec, imgs, imgs)


def _conv_flat_kernel(w_ref, b_ref, cur_ref, nxt_ref, out_ref):
    out_ref[0] = _conv_taps(w_ref, b_ref, cur_ref, nxt_ref)


def _run_conv_flat(imgs, wt, bias):
    return pl.pallas_call(
        _conv_flat_kernel,
        grid=(1, NB),
        in_specs=[
            pl.BlockSpec((1, 9, C, C), lambda g, i: (g, 0, 0, 0)),
            pl.BlockSpec((1, 1, C), lambda g, i: (g, 0, 0)),
            pl.BlockSpec((1, C, BAND), lambda g, i: (g, 0, i)),
            pl.BlockSpec((1, C, BAND), lambda g, i: (g, 0, i + 1)),
        ],
        out_specs=pl.BlockSpec((1, C, BAND), lambda g, i: (g, 0, i)),
        out_shape=jax.ShapeDtypeStruct((1, C, NB * BAND), jnp.float32),
    )(wt, bias, imgs, imgs)


def _score_kernel(s_ref, idx_ref):
    sxv = s_ref[0]
    syv = s_ref[1]
    ones = jnp.ones((1, HWB), jnp.float32)
    sy_mean = lax.dot_general(ones, syv, (((1,), (1,)), ((), ())),
                              preferred_element_type=jnp.float32) / HWB
    lane = lax.broadcasted_iota(jnp.int32, (NBLK, NBLK), 1)

    def pair_body(l, sc):
        syl = s_ref[1, pl.ds(l, 1), :]
        m = jnp.max(sxv + syl, axis=1, keepdims=True)
        return jnp.where(lane == l, m, sc)

    score = lax.fori_loop(0, NBLK, pair_body,
                          jnp.full((NBLK, NBLK), -1e30, jnp.float32))
    score = score + sy_mean
    lane16 = lax.broadcasted_iota(jnp.int32, (NBLK, 16), 1)

    def top_body(k, carry):
        sc, idxs = carry
        mx = jnp.max(sc, axis=1, keepdims=True)
        am = jnp.min(jnp.where(sc == mx, lane, NBLK), axis=1, keepdims=True)
        sc = jnp.where(lane == am, -1e30, sc)
        idxs = jnp.where(lane16 == k, am, idxs)
        return sc, idxs

    _, idxs = lax.fori_loop(0, TOPK, top_body,
                            (score, jnp.zeros((NBLK, 16), jnp.int32)))
    idx_ref[...] = idxs


def _gsum_sc(ytab, idx16):
    mesh = plsc.VectorSubcoreMesh(core_axis_name="c", subcore_axis_name="s")
    ntask = 2 * NCH

    @functools.partial(
        pl.kernel,
        mesh=mesh,
        out_type=jax.ShapeDtypeStruct((NBLK, C, HWB), jnp.float32),
        scratch_types=[
            pltpu.VMEM((NBLK, 16), jnp.int32),
            pltpu.VMEM((2, 16), jnp.int32),
            pltpu.VMEM((2, TOPK, SL, 128), jnp.float32),
            pltpu.VMEM((2, 8, HWB), jnp.float32),
            pltpu.SemaphoreType.DMA,
            pltpu.SemaphoreType.DMA,
            pltpu.SemaphoreType.DMA,
            pltpu.SemaphoreType.DMA,
        ],
    )
    def gsum(ytab_hbm, idx_hbm, out_hbm, idxall, iv, buf, acc,
             sem0, sem1, osem0, osem1):
        wid = lax.axis_index("s") * 2 + lax.axis_index("c")
        pltpu.sync_copy(idx_hbm, idxall)
        ivecs = [idxall[wid * 2 + r, :] * NCH for r in range(2)]
        sems = [sem0, sem1]
        osems = [osem0, osem1]

        def launch(t, slot):
            r, c = divmod(t, NCH)
            iv[slot, :] = ivecs[r] + c
            return pltpu.async_copy(
                ytab_hbm.at[iv.at[slot, pl.ds(0, TOPK)]],
                buf.at[slot], sems[slot])

        handles = {0: launch(0, 0), 1: launch(1, 1)}
        ohandles = {}
        for t in range(ntask):
            slot = t % 2
            r, c = divmod(t, NCH)
            handles.pop(t).wait()

            def add_body(tt, carry):
                ch = tt // 49
                col = (tt % 49) * 16
                s = (tt * 16) // 128
                o = (tt * 16) % 128
                v = (buf[slot, 0, s, pl.ds(o, 16)]
                     + buf[slot, 1, s, pl.ds(o, 16)]
                     + buf[slot, 2, s, pl.ds(o, 16)]
                     + buf[slot, 3, s, pl.ds(o, 16)]
                     + buf[slot, 4, s, pl.ds(o, 16)])
                acc[slot, ch, pl.ds(col, 16)] = v
                return carry

            if t >= 2:
                ohandles.pop(t - 2).wait()
            lax.fori_loop(0, CH // 16, add_body, 0, unroll=4)
            ohandles[t] = pltpu.async_copy(
                acc.at[slot], out_hbm.at[wid * 2 + r, pl.ds(c * 8, 8)],
                osems[slot])
            if t + 2 < ntask:
                handles[t + 2] = launch(t + 2, slot)
        ohandles.pop(ntask - 2).wait()
        ohandles.pop(ntask - 1).wait()

    return gsum(ytab, idx16)


def _mm_kernel(xy_ref, ys_ref, out_ref):
    for b in range(4):
        xb = xy_ref[0, b]
        g = lax.dot_general(xb, ys_ref[b], (((1,), (1,)), ((), ())),
                            preferred_element_type=jnp.float32)
        out_ref[b] = lax.dot_general(g, xb, (((1,), (0,)), ((), ())),
                                     preferred_element_type=jnp.float32)


def _mm_tc(xyblk, ysum3):
    return pl.pallas_call(
        _mm_kernel,
        grid=(NBLK // 4,),
        in_specs=[
            pl.BlockSpec((1, 4, C, HWB), lambda b4: (0, b4, 0, 0)),
            pl.BlockSpec((4, C, HWB), lambda b4: (b4, 0, 0)),
        ],
        out_specs=pl.BlockSpec((4, C, HWB), lambda b4: (b4, 0, 0)),
        out_shape=jax.ShapeDtypeStruct((NBLK, C, HWB), jnp.float32),
    )(xyblk, ysum3)


def _pad_flat(img):
    p = jnp.pad(img, ((0, 0), (0, 0), (1, RP - H - 1), (1, WP - W - 1)))
    return p.reshape(img.shape[0], C, RP * WP)


def kernel(x, y, xconv_w, xconv_b, yconv_w, yconv_b, out_w, out_b, ln_w, ln_b):
    imgs = _pad_flat(jnp.stack([x[0], y[0]]))
    wt = jnp.stack([xconv_w, yconv_w]).transpose(0, 3, 4, 1, 2).reshape(2, 9, C, C)
    bias = jnp.stack([xconv_b, yconv_b]).reshape(2, 1, C)
    wvec = jnp.stack([ln_w[0, C:], ln_w[0, :C]]).reshape(2, 1, C)

    xyblk, sblk = _run_conv_blk(imgs, wt, bias, wvec)

    idx16 = pl.pallas_call(
        _score_kernel,
        out_shape=jax.ShapeDtypeStruct((NBLK, 16), jnp.int32),
    )(sblk)

    ysum = _gsum_sc(xyblk[1].reshape(NBLK * NCH, SL, 128), idx16)
    ob = _mm_tc(xyblk, ysum)

    oimg = ob.reshape(NB, NB, C, BS, BS).transpose(2, 0, 3, 1, 4).reshape(1, C, H, W)
    wt3 = out_w.transpose(2, 3, 0, 1).reshape(1, 9, C, C)
    conv3 = _run_conv_flat(_pad_flat(oimg), wt3, out_b.reshape(1, 1, C))
    return conv3.reshape(1, C, H, WP)[:, :, :, 1:W + 1]


# speedup vs baseline: 11.0060x; 1.0076x over previous
import functools

import jax
import jax.numpy as jnp
from jax import lax
from jax.experimental import pallas as pl
from jax.experimental.pallas import tpu as pltpu
from jax.experimental.pallas import tpu_sc as plsc

C = 64
H = W = 224
BS = 28
NB = 8
NBLK = NB * NB
HWB = BS * BS
TOPK = 5
WP = 256
RP = 252
BAND = BS * WP
ROWF = C * HWB
NCH = 8
CH = ROWF // NCH
SL = CH // 128


def _conv_taps(w_ref, b_ref, cur_ref, nxt_ref):
    xcat = jnp.concatenate([cur_ref[0], nxt_ref[0]], axis=1)
    acc = jnp.zeros((C, BAND), jnp.float32)
    for dy in range(3):
        win = xcat[:, dy * WP: dy * WP + BAND]
        for dx in (-1, 0, 1):
            xs = win if dx == 0 else jnp.roll(win, -dx, axis=1)
            acc = acc + jnp.dot(w_ref[0, dy * 3 + dx + 1], xs,
                                preferred_element_type=jnp.float32)
    return jnp.maximum(acc + b_ref[0, 0][:, None], 0.0)


def _conv_blk_kernel(w_ref, b_ref, wv_ref, cur_ref, nxt_ref, out_ref, s_ref):
    acc = _conv_taps(w_ref, b_ref, cur_ref, nxt_ref)
    acc = jnp.roll(acc, -1, axis=1)
    s = jnp.dot(wv_ref[0], acc, preferred_element_type=jnp.float32)
    acc3 = acc.reshape(C, BS, WP)
    s3 = s.reshape(1, BS, WP)
    for j in range(NB):
        out_ref[0, j] = acc3[:, :, BS * j: BS * (j + 1)].reshape(C, HWB)
        s_ref[0, j] = s3[:, :, BS * j: BS * (j + 1)].reshape(HWB)


def _run_conv_blk(imgs, wt, bias, wvec):
    return pl.pallas_call(
        _conv_blk_kernel,
        grid=(2, NB),
        in_specs=[
            pl.BlockSpec((1, 9, C, C), lambda g, i: (g, 0, 0, 0)),
            pl.BlockSpec((1, 1, C), lambda g, i: (g, 0, 0)),
            pl.BlockSpec((1, 1, C), lambda g, i: (g, 0, 0)),
            pl.BlockSpec((1, C, BAND), lambda g, i: (g, 0, i)),
            pl.BlockSpec((1, C, BAND), lambda g, i: (g, 0, i + 1)),
        ],
        out_specs=[
            pl.BlockSpec((1, NB, C, HWB), lambda g, i: (g, i, 0, 0)),
            pl.BlockSpec((1, NB, HWB), lambda g, i: (g, i, 0)),
        ],
        out_shape=[
            jax.ShapeDtypeStruct((2, NBLK, C, HWB), jnp.float32),
            jax.ShapeDtypeStruct((2, NBLK, HWB), jnp.float32),
        ],
    )(wt, bias, wvec, imgs, imgs)


def _conv_flat_kernel(w_ref, b_ref, cur_ref, nxt_ref, out_ref):
    out_ref[0] = _conv_taps(w_ref, b_ref, cur_ref, nxt_ref)


def _run_conv_flat(imgs, wt, bias):
    return pl.pallas_call(
        _conv_flat_kernel,
        grid=(1, NB),
        in_specs=[
            pl.BlockSpec((1, 9, C, C), lambda g, i: (g, 0, 0, 0)),
            pl.BlockSpec((1, 1, C), lambda g, i: (g, 0, 0)),
            pl.BlockSpec((1, C, BAND), lambda g, i: (g, 0, i)),
            pl.BlockSpec((1, C, BAND), lambda g, i: (g, 0, i + 1)),
        ],
        out_specs=pl.BlockSpec((1, C, BAND), lambda g, i: (g, 0, i)),
        out_shape=jax.ShapeDtypeStruct((1, C, NB * BAND), jnp.float32),
    )(wt, bias, imgs, imgs)


def _score_kernel(s_ref, idx_ref):
    sxv = s_ref[0]
    syv = s_ref[1]
    ones = jnp.ones((1, HWB), jnp.float32)
    sy_mean = lax.dot_general(ones, syv, (((1,), (1,)), ((), ())),
                              preferred_element_type=jnp.float32) / HWB
    lane = lax.broadcasted_iota(jnp.int32, (NBLK, NBLK), 1)

    def pair_body(l, sc):
        syl = s_ref[1, pl.ds(l, 1), :]
        m = jnp.max(sxv + syl, axis=1, keepdims=True)
        return jnp.where(lane == l, m, sc)

    score = lax.fori_loop(0, NBLK, pair_body,
                          jnp.full((NBLK, NBLK), -1e30, jnp.float32))
    score = score + sy_mean
    lane16 = lax.broadcasted_iota(jnp.int32, (NBLK, 16), 1)

    def top_body(k, carry):
        sc, idxs = carry
        mx = jnp.max(sc, axis=1, keepdims=True)
        am = jnp.min(jnp.where(sc == mx, lane, NBLK), axis=1, keepdims=True)
        sc = jnp.where(lane == am, -1e30, sc)
        idxs = jnp.where(lane16 == k, am, idxs)
        return sc, idxs

    _, idxs = lax.fori_loop(0, TOPK, top_body,
                            (score, jnp.zeros((NBLK, 16), jnp.int32)))
    idx_ref[...] = idxs


def _gsum_sc(ytab, idx16):
    mesh = plsc.VectorSubcoreMesh(core_axis_name="c", subcore_axis_name="s")
    ntask = 2 * NCH

    @functools.partial(
        pl.kernel,
        mesh=mesh,
        out_type=jax.ShapeDtypeStruct((NBLK, C, HWB), jnp.float32),
        scratch_types=[
            pltpu.VMEM((NBLK, 16), jnp.int32),
            pltpu.VMEM((2, 16), jnp.int32),
            pltpu.VMEM((2, TOPK, SL, 128), jnp.float32),
            pltpu.VMEM((2, 8, HWB), jnp.float32),
            pltpu.SemaphoreType.DMA,
            pltpu.SemaphoreType.DMA,
            pltpu.SemaphoreType.DMA,
            pltpu.SemaphoreType.DMA,
        ],
    )
    def gsum(ytab_hbm, idx_hbm, out_hbm, idxall, iv, buf, acc,
             sem0, sem1, osem0, osem1):
        wid = lax.axis_index("s") * 2 + lax.axis_index("c")
        pltpu.sync_copy(idx_hbm, idxall)
        ivecs = [idxall[wid * 2 + r, :] * NCH for r in range(2)]
        sems = [sem0, sem1]
        osems = [osem0, osem1]

        def launch(t, slot):
            r, c = divmod(t, NCH)
            iv[slot, :] = ivecs[r] + c
            return pltpu.async_copy(
                ytab_hbm.at[iv.at[slot, pl.ds(0, TOPK)]],
                buf.at[slot], sems[slot])

        handles = {0: launch(0, 0), 1: launch(1, 1)}
        ohandles = {}
        for t in range(ntask):
            slot = t % 2
            r, c = divmod(t, NCH)
            handles.pop(t).wait()

            def add_body(tt, carry):
                ch = tt // 49
                col = (tt % 49) * 16
                s = (tt * 16) // 128
                o = (tt * 16) % 128
                v = (buf[slot, 0, s, pl.ds(o, 16)]
                     + buf[slot, 1, s, pl.ds(o, 16)]
                     + buf[slot, 2, s, pl.ds(o, 16)]
                     + buf[slot, 3, s, pl.ds(o, 16)]
                     + buf[slot, 4, s, pl.ds(o, 16)])
                acc[slot, ch, pl.ds(col, 16)] = v
                return carry

            if t >= 2:
                ohandles.pop(t - 2).wait()
            lax.fori_loop(0, CH // 16, add_body, 0, unroll=4)
            ohandles[t] = pltpu.async_copy(
                acc.at[slot], out_hbm.at[wid * 2 + r, pl.ds(c * 8, 8)],
                osems[slot])
            if t + 2 < ntask:
                handles[t + 2] = launch(t + 2, slot)
        ohandles.pop(ntask - 2).wait()
        ohandles.pop(ntask - 1).wait()

    return gsum(ytab, idx16)


def _mm_kernel(xy_ref, ys_ref, out_ref):
    for b in range(8):
        xb = xy_ref[0, b]
        g = lax.dot_general(xb, ys_ref[b], (((1,), (1,)), ((), ())),
                            preferred_element_type=jnp.float32)
        out_ref[b] = lax.dot_general(g, xb, (((1,), (0,)), ((), ())),
                                     preferred_element_type=jnp.float32)


def _mm_tc(xyblk, ysum3):
    return pl.pallas_call(
        _mm_kernel,
        grid=(NBLK // 8,),
        in_specs=[
            pl.BlockSpec((1, 8, C, HWB), lambda b8: (0, b8, 0, 0)),
            pl.BlockSpec((8, C, HWB), lambda b8: (b8, 0, 0)),
        ],
        out_specs=pl.BlockSpec((8, C, HWB), lambda b8: (b8, 0, 0)),
        out_shape=jax.ShapeDtypeStruct((NBLK, C, HWB), jnp.float32),
    )(xyblk, ysum3)


def _pad_flat(img):
    p = jnp.pad(img, ((0, 0), (0, 0), (1, RP - H - 1), (1, WP - W - 1)))
    return p.reshape(img.shape[0], C, RP * WP)


def kernel(x, y, xconv_w, xconv_b, yconv_w, yconv_b, out_w, out_b, ln_w, ln_b):
    imgs = _pad_flat(jnp.stack([x[0], y[0]]))
    wt = jnp.stack([xconv_w, yconv_w]).transpose(0, 3, 4, 1, 2).reshape(2, 9, C, C)
    bias = jnp.stack([xconv_b, yconv_b]).reshape(2, 1, C)
    wvec = jnp.stack([ln_w[0, C:], ln_w[0, :C]]).reshape(2, 1, C)

    xyblk, sblk = _run_conv_blk(imgs, wt, bias, wvec)

    idx16 = pl.pallas_call(
        _score_kernel,
        out_shape=jax.ShapeDtypeStruct((NBLK, 16), jnp.int32),
    )(sblk)

    ysum = _gsum_sc(xyblk[1].reshape(NBLK * NCH, SL, 128), idx16)
    ob = _mm_tc(xyblk, ysum)

    oimg = ob.reshape(NB, NB, C, BS, BS).transpose(2, 0, 3, 1, 4).reshape(1, C, H, W)
    wt3 = out_w.transpose(2, 3, 0, 1).reshape(1, 9, C, C)
    conv3 = _run_conv_flat(_pad_flat(oimg), wt3, out_b.reshape(1, 1, C))
    return conv3.reshape(1, C, H, WP)[:, :, :, 1:W + 1]
